# baseline (speedup 1.0000x reference)
import jax
import jax.numpy as jnp
from jax.experimental import pallas as pl
from jax.experimental.pallas import tpu as pltpu

_NE = 64
_CAP = 128
_T = 128
_NW = _NE * _CAP // 32


def _gate_kernel(ids_ref, valid_ref, combine_ref, words_ref, counts_ref):
    g = pl.program_id(0)

    @pl.when(g == 0)
    def _():
        for e in range(_NE):
            counts_ref[e] = 0

    e_iota = jax.lax.broadcasted_iota(jnp.int32, (_NE, _CAP), 0)
    c_iota = jax.lax.broadcasted_iota(jnp.int32, (_NE, _CAP), 1)
    flat_iota = e_iota * _CAP + c_iota
    w_iota = jax.lax.broadcasted_iota(jnp.int32, (1, _NW), 1)

    def body(i, _):
        t = g * _T + i
        e = ids_ref[t]
        v = valid_ref[t]
        cnt = counts_ref[e]
        counts_ref[e] = cnt + v
        kept = (v == 1) & (cnt < _CAP)
        tgt = jnp.where(kept, e * _CAP + cnt, -1)
        slab = flat_iota == tgt
        combine_ref[i] = slab.astype(jnp.float32)
        word = jnp.where(w_iota == (tgt >> 5), 1 << (tgt & 31), 0)
        words_ref[pl.ds(i, 1), :] = word
        return 0

    jax.lax.fori_loop(0, _T, body, 0)


def kernel(input, mask, domain_ids):
    n_tokens = input.shape[0]
    grid = n_tokens // _T
    ids = domain_ids.astype(jnp.int32)
    valid = jnp.logical_not(mask).astype(jnp.int32)

    combine, words = pl.pallas_call(
        _gate_kernel,
        grid=(grid,),
        in_specs=[
            pl.BlockSpec(memory_space=pltpu.SMEM),
            pl.BlockSpec(memory_space=pltpu.SMEM),
        ],
        out_specs=[
            pl.BlockSpec((_T, _NE, _CAP), lambda g: (g, 0, 0)),
            pl.BlockSpec((_T, _NW), lambda g: (g, 0)),
        ],
        out_shape=[
            jax.ShapeDtypeStruct((n_tokens, _NE, _CAP), jnp.float32),
            jax.ShapeDtypeStruct((n_tokens, _NW), jnp.int32),
        ],
        scratch_shapes=[pltpu.SMEM((_NE,), jnp.int32)],
    )(ids, valid)

    w4 = words.reshape(n_tokens, _NE, _CAP // 32, 1)
    bits = jax.lax.broadcasted_iota(jnp.int32, (1, 1, 1, 32), 3)
    dispatch = ((w4 >> bits) & 1).astype(jnp.bool_).reshape(n_tokens, _NE, _CAP)

    l_aux = jnp.zeros((), dtype=jnp.float32)
    return (l_aux, combine, dispatch)


# speedup vs baseline: 1.4289x; 1.4289x over previous
import jax
import jax.numpy as jnp
from jax.experimental import pallas as pl
from jax.experimental.pallas import tpu as pltpu

_NE = 64
_CAP = 128
_T = 128


def _gate_kernel(ids_ref, valid_ref, combine_ref, wlo_ref, whi_ref, counts_ref):
    g = pl.program_id(0)

    @pl.when(g == 0)
    def _():
        for e in range(_NE):
            counts_ref[e] = 0

    e_iota = jax.lax.broadcasted_iota(jnp.int32, (_NE, _CAP), 0)
    c_iota = jax.lax.broadcasted_iota(jnp.int32, (_NE, _CAP), 1)
    flat_iota = e_iota * _CAP + c_iota
    c_row = jax.lax.broadcasted_iota(jnp.int32, (1, _CAP), 1)

    def body(i, _):
        t = g * _T + i
        e = ids_ref[t]
        v = valid_ref[t]
        cnt = counts_ref[e]
        counts_ref[e] = cnt + v
        kept = (v == 1) & (cnt < _CAP)
        tgt = jnp.where(kept, e * _CAP + cnt, -1)
        slab = flat_iota == tgt
        combine_ref[i] = slab.astype(jnp.float32)
        bit = jnp.where(kept, 1 << (e & 31), 0)
        onehot = c_row == jnp.where(kept, cnt, -1)
        wlo_ref[pl.ds(i, 1), :] = jnp.where(
            onehot, jnp.where(e < 32, bit, 0), 0)
        whi_ref[pl.ds(i, 1), :] = jnp.where(
            onehot, jnp.where(e >= 32, bit, 0), 0)
        return 0

    jax.lax.fori_loop(0, _T, body, 0)


def kernel(input, mask, domain_ids):
    n_tokens = input.shape[0]
    grid = n_tokens // _T
    ids = domain_ids.astype(jnp.int32)
    valid = jnp.logical_not(mask).astype(jnp.int32)

    combine, wlo, whi = pl.pallas_call(
        _gate_kernel,
        grid=(grid,),
        in_specs=[
            pl.BlockSpec(memory_space=pltpu.SMEM),
            pl.BlockSpec(memory_space=pltpu.SMEM),
        ],
        out_specs=[
            pl.BlockSpec((_T, _NE, _CAP), lambda g: (g, 0, 0)),
            pl.BlockSpec((_T, _CAP), lambda g: (g, 0)),
            pl.BlockSpec((_T, _CAP), lambda g: (g, 0)),
        ],
        out_shape=[
            jax.ShapeDtypeStruct((n_tokens, _NE, _CAP), jnp.float32),
            jax.ShapeDtypeStruct((n_tokens, _CAP), jnp.int32),
            jax.ShapeDtypeStruct((n_tokens, _CAP), jnp.int32),
        ],
        scratch_shapes=[pltpu.SMEM((_NE,), jnp.int32)],
    )(ids, valid)

    bits = jax.lax.broadcasted_iota(jnp.int32, (1, 32, 1), 1)
    dlo = (wlo[:, None, :] >> bits) & 1
    dhi = (whi[:, None, :] >> bits) & 1
    dispatch = jnp.concatenate([dlo, dhi], axis=1).astype(jnp.bool_)

    l_aux = jnp.zeros((), dtype=jnp.float32)
    return (l_aux, combine, dispatch)


# speedup vs baseline: 2.2653x; 1.5853x over previous
import jax
import jax.numpy as jnp
from jax.experimental import pallas as pl
from jax.experimental.pallas import tpu as pltpu

_NE = 64
_CAP = 128
_T = 128


def _gate_kernel(ids_ref, valid_ref, combine_ref, wlo_ref, whi_ref, counts_ref):
    g = pl.program_id(0)

    @pl.when(g == 0)
    def _():
        for e in range(_NE):
            counts_ref[e] = 0

    e_iota = jax.lax.broadcasted_iota(jnp.int32, (_NE, _CAP), 0)
    c_iota = jax.lax.broadcasted_iota(jnp.int32, (_NE, _CAP), 1)
    flat_iota = e_iota * _CAP + c_iota
    c_row = jax.lax.broadcasted_iota(jnp.int32, (1, _CAP), 1)

    def body(i, _):
        t = g * _T + i
        e = ids_ref[t]
        v = valid_ref[t]
        cnt = counts_ref[e]
        counts_ref[e] = cnt + v
        kept = (v == 1) & (cnt < _CAP)
        tgt = jnp.where(kept, e * _CAP + cnt, -1)
        slab = flat_iota == tgt
        combine_ref[i] = slab.astype(jnp.float32)
        bit = jnp.where(kept, 1 << (e & 31), 0)
        onehot = c_row == jnp.where(kept, cnt, -1)
        wlo_ref[pl.ds(i, 1), :] = jnp.where(
            onehot, jnp.where(e < 32, bit, 0), 0)
        whi_ref[pl.ds(i, 1), :] = jnp.where(
            onehot, jnp.where(e >= 32, bit, 0), 0)
        return 0

    jax.lax.fori_loop(0, _T, body, 0)


def kernel(input, mask, domain_ids):
    n_tokens = input.shape[0]
    grid = n_tokens // _T
    ids = domain_ids.astype(jnp.int32)
    valid = jnp.logical_not(mask).astype(jnp.int32)

    combine, wlo, whi = pl.pallas_call(
        _gate_kernel,
        grid=(grid,),
        in_specs=[
            pl.BlockSpec(memory_space=pltpu.SMEM),
            pl.BlockSpec(memory_space=pltpu.SMEM),
        ],
        out_specs=[
            pl.BlockSpec((_T, _NE, _CAP), lambda g: (g, 0, 0)),
            pl.BlockSpec((_T, _CAP), lambda g: (g, 0)),
            pl.BlockSpec((_T, _CAP), lambda g: (g, 0)),
        ],
        out_shape=[
            jax.ShapeDtypeStruct((n_tokens, _NE, _CAP), jnp.float32),
            jax.ShapeDtypeStruct((n_tokens, _CAP), jnp.int32),
            jax.ShapeDtypeStruct((n_tokens, _CAP), jnp.int32),
        ],
        scratch_shapes=[pltpu.SMEM((_NE,), jnp.int32)],
    )(ids, valid)

    e3 = jax.lax.broadcasted_iota(jnp.int32, (1, _NE, 1), 1)
    wsel = jnp.where(e3 < 32, wlo[:, None, :], whi[:, None, :])
    dispatch = ((wsel >> (e3 & 31)) & 1).astype(jnp.bool_)

    l_aux = jnp.zeros((), dtype=jnp.float32)
    return (l_aux, combine, dispatch)


# speedup vs baseline: 2.6267x; 1.1595x over previous
import jax
import jax.numpy as jnp
from jax.experimental import pallas as pl
from jax.experimental.pallas import tpu as pltpu

_NE = 64
_CAP = 128
_T = 128


def _gate_kernel(ids_ref, valid_ref, combine_ref, tgt_ref, counts_ref):
    g = pl.program_id(0)

    @pl.when(g == 0)
    def _():
        for e in range(_NE):
            counts_ref[e] = 0

    e_iota = jax.lax.broadcasted_iota(jnp.int32, (_NE, _CAP), 0)
    c_iota = jax.lax.broadcasted_iota(jnp.int32, (_NE, _CAP), 1)
    flat_iota = e_iota * _CAP + c_iota

    def body(i, _):
        t = g * _T + i
        e = ids_ref[t]
        v = valid_ref[t]
        cnt = counts_ref[e]
        counts_ref[e] = cnt + v
        kept = (v == 1) & (cnt < _CAP)
        tgt = jnp.where(kept, e * _CAP + cnt, -1)
        tgt_ref[t] = tgt
        combine_ref[i] = (flat_iota == tgt).astype(jnp.float32)
        return 0

    jax.lax.fori_loop(0, _T, body, 0)


def kernel(input, mask, domain_ids):
    n_tokens = input.shape[0]
    grid = n_tokens // _T
    ids = domain_ids.astype(jnp.int32)
    valid = jnp.logical_not(mask).astype(jnp.int32)

    combine, target = pl.pallas_call(
        _gate_kernel,
        grid=(grid,),
        in_specs=[
            pl.BlockSpec(memory_space=pltpu.SMEM),
            pl.BlockSpec(memory_space=pltpu.SMEM),
        ],
        out_specs=[
            pl.BlockSpec((_T, _NE, _CAP), lambda g: (g, 0, 0)),
            pl.BlockSpec(memory_space=pltpu.SMEM),
        ],
        out_shape=[
            jax.ShapeDtypeStruct((n_tokens, _NE, _CAP), jnp.float32),
            jax.ShapeDtypeStruct((n_tokens,), jnp.int32),
        ],
        scratch_shapes=[pltpu.SMEM((_NE,), jnp.int32)],
    )(ids, valid)

    e3 = jax.lax.broadcasted_iota(jnp.int32, (1, _NE, 1), 1)
    c3 = jax.lax.broadcasted_iota(jnp.int32, (1, 1, _CAP), 2)
    dispatch = (target[:, None, None] == e3 * _CAP + c3)

    l_aux = jnp.zeros((), dtype=jnp.float32)
    return (l_aux, combine, dispatch)


# speedup vs baseline: 3.1511x; 1.1996x over previous
import jax
import jax.numpy as jnp
from jax.experimental import pallas as pl
from jax.experimental.pallas import tpu as pltpu

_NE = 64
_CAP = 128
_T = 128


def _gate_kernel(ids_ref, valid_ref, combine_ref, tgt_ref, counts_ref):
    g = pl.program_id(0)

    @pl.when(g == 0)
    def _():
        for e in range(_NE):
            counts_ref[e] = 0

    e_iota = jax.lax.broadcasted_iota(jnp.int32, (_NE, _CAP), 0)
    c_iota = jax.lax.broadcasted_iota(jnp.int32, (_NE, _CAP), 1)
    flat_iota = e_iota * _CAP + c_iota

    def body(i, _):
        t = g * _T + i
        e = ids_ref[t]
        v = valid_ref[t]
        cnt = counts_ref[e]
        counts_ref[e] = cnt + v
        kept = (v == 1) & (cnt < _CAP)
        tgt = jnp.where(kept, e * _CAP + cnt, -1)
        tgt_ref[t] = tgt
        combine_ref[i] = (flat_iota == tgt).astype(jnp.float32)
        return 0

    jax.lax.fori_loop(0, _T, body, 0, unroll=4)


def kernel(input, mask, domain_ids):
    n_tokens = input.shape[0]
    grid = n_tokens // _T
    ids = domain_ids.astype(jnp.int32)
    valid = jnp.logical_not(mask).astype(jnp.int32)

    combine, target = pl.pallas_call(
        _gate_kernel,
        grid=(grid,),
        in_specs=[
            pl.BlockSpec(memory_space=pltpu.SMEM),
            pl.BlockSpec(memory_space=pltpu.SMEM),
        ],
        out_specs=[
            pl.BlockSpec((_T, _NE, _CAP), lambda g: (g, 0, 0)),
            pl.BlockSpec(memory_space=pltpu.SMEM),
        ],
        out_shape=[
            jax.ShapeDtypeStruct((n_tokens, _NE, _CAP), jnp.float32),
            jax.ShapeDtypeStruct((n_tokens,), jnp.int32),
        ],
        scratch_shapes=[pltpu.SMEM((_NE,), jnp.int32)],
    )(ids, valid)

    e3 = jax.lax.broadcasted_iota(jnp.int32, (1, _NE, 1), 1)
    c3 = jax.lax.broadcasted_iota(jnp.int32, (1, 1, _CAP), 2)
    dispatch = (target[:, None, None] == e3 * _CAP + c3)

    l_aux = jnp.zeros((), dtype=jnp.float32)
    return (l_aux, combine, dispatch)


# speedup vs baseline: 3.2740x; 1.0390x over previous
import jax
import jax.numpy as jnp
from jax.experimental import pallas as pl
from jax.experimental.pallas import tpu as pltpu

_NE = 64
_CAP = 128
_T = 128


def _gate_kernel(ids_ref, valid_ref, combine_ref, tgt_ref, counts_ref):
    g = pl.program_id(0)

    @pl.when(g == 0)
    def _():
        for e in range(_NE):
            counts_ref[e] = 0

    e_iota = jax.lax.broadcasted_iota(jnp.int32, (_NE, _CAP), 0)
    c_iota = jax.lax.broadcasted_iota(jnp.int32, (_NE, _CAP), 1)
    flat_iota = e_iota * _CAP + c_iota

    def body(i, _):
        t = g * _T + i
        e = ids_ref[t]
        v = valid_ref[t]
        cnt = counts_ref[e]
        counts_ref[e] = cnt + v
        kept = (v == 1) & (cnt < _CAP)
        tgt = jnp.where(kept, e * _CAP + cnt, -1)
        tgt_ref[t] = tgt
        combine_ref[i] = (flat_iota == tgt).astype(jnp.float32)
        return 0

    jax.lax.fori_loop(0, _T, body, 0, unroll=8)


def kernel(input, mask, domain_ids):
    n_tokens = input.shape[0]
    grid = n_tokens // _T
    ids = domain_ids.astype(jnp.int32)
    valid = jnp.logical_not(mask).astype(jnp.int32)

    combine, target = pl.pallas_call(
        _gate_kernel,
        grid=(grid,),
        in_specs=[
            pl.BlockSpec(memory_space=pltpu.SMEM),
            pl.BlockSpec(memory_space=pltpu.SMEM),
        ],
        out_specs=[
            pl.BlockSpec((_T, _NE, _CAP), lambda g: (g, 0, 0)),
            pl.BlockSpec(memory_space=pltpu.SMEM),
        ],
        out_shape=[
            jax.ShapeDtypeStruct((n_tokens, _NE, _CAP), jnp.float32),
            jax.ShapeDtypeStruct((n_tokens,), jnp.int32),
        ],
        scratch_shapes=[pltpu.SMEM((_NE,), jnp.int32)],
    )(ids, valid)

    e3 = jax.lax.broadcasted_iota(jnp.int32, (1, _NE, 1), 1)
    c3 = jax.lax.broadcasted_iota(jnp.int32, (1, 1, _CAP), 2)
    dispatch = (target[:, None, None] == e3 * _CAP + c3)

    l_aux = jnp.zeros((), dtype=jnp.float32)
    return (l_aux, combine, dispatch)


# speedup vs baseline: 3.3301x; 1.0171x over previous
import jax
import jax.numpy as jnp
from jax.experimental import pallas as pl
from jax.experimental.pallas import tpu as pltpu

_NE = 64
_CAP = 128
_T = 128


def _gate_kernel(ids_ref, valid_ref, combine_ref, tgt_ref, counts_ref):
    g = pl.program_id(0)

    @pl.when(g == 0)
    def _():
        for e in range(_NE):
            counts_ref[e] = 0

    e_iota = jax.lax.broadcasted_iota(jnp.int32, (_NE, _CAP), 0)
    c_iota = jax.lax.broadcasted_iota(jnp.int32, (_NE, _CAP), 1)
    flat_iota = e_iota * _CAP + c_iota

    def body(i, _):
        t = g * _T + i
        e = ids_ref[t]
        v = valid_ref[t]
        cnt = counts_ref[e]
        counts_ref[e] = cnt + v
        kept = (v == 1) & (cnt < _CAP)
        tgt = jnp.where(kept, e * _CAP + cnt, -1)
        tgt_ref[t] = tgt
        combine_ref[i] = (flat_iota == tgt).astype(jnp.float32)
        return 0

    jax.lax.fori_loop(0, _T, body, 0, unroll=16)


def kernel(input, mask, domain_ids):
    n_tokens = input.shape[0]
    grid = n_tokens // _T
    ids = domain_ids.astype(jnp.int32)
    valid = jnp.logical_not(mask).astype(jnp.int32)

    combine, target = pl.pallas_call(
        _gate_kernel,
        grid=(grid,),
        in_specs=[
            pl.BlockSpec(memory_space=pltpu.SMEM),
            pl.BlockSpec(memory_space=pltpu.SMEM),
        ],
        out_specs=[
            pl.BlockSpec((_T, _NE, _CAP), lambda g: (g, 0, 0)),
            pl.BlockSpec(memory_space=pltpu.SMEM),
        ],
        out_shape=[
            jax.ShapeDtypeStruct((n_tokens, _NE, _CAP), jnp.float32),
            jax.ShapeDtypeStruct((n_tokens,), jnp.int32),
        ],
        scratch_shapes=[pltpu.SMEM((_NE,), jnp.int32)],
    )(ids, valid)

    e3 = jax.lax.broadcasted_iota(jnp.int32, (1, _NE, 1), 1)
    c3 = jax.lax.broadcasted_iota(jnp.int32, (1, 1, _CAP), 2)
    dispatch = (target[:, None, None] == e3 * _CAP + c3)

    l_aux = jnp.zeros((), dtype=jnp.float32)
    return (l_aux, combine, dispatch)


# speedup vs baseline: 3.3917x; 1.0185x over previous
import jax
import jax.numpy as jnp
from jax.experimental import pallas as pl
from jax.experimental.pallas import tpu as pltpu

_NE = 64
_CAP = 128
_T = 256


def _gate_kernel(ids_ref, valid_ref, combine_ref, tgt_ref, counts_ref):
    g = pl.program_id(0)

    @pl.when(g == 0)
    def _():
        for e in range(_NE):
            counts_ref[e] = 0

    e_iota = jax.lax.broadcasted_iota(jnp.int32, (_NE, _CAP), 0)
    c_iota = jax.lax.broadcasted_iota(jnp.int32, (_NE, _CAP), 1)
    flat_iota = e_iota * _CAP + c_iota

    def body(i, _):
        t = g * _T + i
        e = ids_ref[t]
        v = valid_ref[t]
        cnt = counts_ref[e]
        counts_ref[e] = cnt + v
        kept = (v == 1) & (cnt < _CAP)
        tgt = jnp.where(kept, e * _CAP + cnt, -1)
        tgt_ref[t] = tgt
        combine_ref[i] = (flat_iota == tgt).astype(jnp.float32)
        return 0

    jax.lax.fori_loop(0, _T, body, 0, unroll=16)


def kernel(input, mask, domain_ids):
    n_tokens = input.shape[0]
    grid = n_tokens // _T
    ids = domain_ids.astype(jnp.int32)
    valid = jnp.logical_not(mask).astype(jnp.int32)

    combine, target = pl.pallas_call(
        _gate_kernel,
        grid=(grid,),
        in_specs=[
            pl.BlockSpec(memory_space=pltpu.SMEM),
            pl.BlockSpec(memory_space=pltpu.SMEM),
        ],
        out_specs=[
            pl.BlockSpec((_T, _NE, _CAP), lambda g: (g, 0, 0)),
            pl.BlockSpec(memory_space=pltpu.SMEM),
        ],
        out_shape=[
            jax.ShapeDtypeStruct((n_tokens, _NE, _CAP), jnp.float32),
            jax.ShapeDtypeStruct((n_tokens,), jnp.int32),
        ],
        scratch_shapes=[pltpu.SMEM((_NE,), jnp.int32)],
    )(ids, valid)

    e3 = jax.lax.broadcasted_iota(jnp.int32, (1, _NE, 1), 1)
    c3 = jax.lax.broadcasted_iota(jnp.int32, (1, 1, _CAP), 2)
    dispatch = (target[:, None, None] == e3 * _CAP + c3)

    l_aux = jnp.zeros((), dtype=jnp.float32)
    return (l_aux, combine, dispatch)
